# baseline (speedup 1.0000x reference)
import functools

import jax
import jax.numpy as jnp
from jax import lax
from jax.experimental import pallas as pl
from jax.experimental.pallas import tpu as pltpu
from jax.experimental.pallas import tpu_sc as plsc

NODES = 10000
FDIM = 128
NC = 2
NS = 16
LANES = 16
CH = 128
RPT = 640
NPAD = NS * RPT
RPTA = 632
NPADA = NS * RPTA
SLOW_CORE = 0
SLOW_NUM, SLOW_DEN = 5, 14
NBUF = 2
NRING = 4

_MESH = plsc.VectorSubcoreMesh(
    core_axis_name="c", subcore_axis_name="s", num_cores=NC, num_subcores=NS)


def _deg_body(nslow, nfast, dst_hbm, out_hbm, dst_v, val_v, zer_v, deg_sh):
    c = lax.axis_index("c")
    s = lax.axis_index("s")
    pltpu.sync_copy(dst_hbm.at[c, s], dst_v)
    one = jnp.ones((LANES,), jnp.float32)
    zero = jnp.zeros((LANES,), jnp.float32)
    for i in range(CH // LANES):
        val_v[pl.ds(i * LANES, LANES)] = one

    def zfill(i, carry):
        zer_v[pl.ds(i * LANES, LANES)] = zero
        return carry

    lax.fori_loop(0, RPT // LANES, zfill, 0)
    pltpu.sync_copy(zer_v, deg_sh.at[pl.ds(s * RPT, RPT)])
    plsc.subcore_barrier()

    nch = jnp.where(c == SLOW_CORE, nslow, nfast)

    def body(j, carry):
        pltpu.sync_copy(val_v, deg_sh.at[dst_v.at[j]], add=True)
        return carry

    lax.fori_loop(0, nch, body, 0)
    plsc.subcore_barrier()
    pltpu.sync_copy(deg_sh.at[pl.ds(s * RPT, RPT)],
                    out_hbm.at[c, pl.ds(s * RPT, RPT)])


@functools.lru_cache(maxsize=None)
def _make_deg(nslow, nfast):
    nmax = max(nslow, nfast)
    return pl.kernel(
        functools.partial(_deg_body, nslow, nfast),
        out_type=jax.ShapeDtypeStruct((NC, NPAD), jnp.float32),
        mesh=_MESH,
        scratch_types=[
            pltpu.VMEM((nmax, CH), jnp.int32),
            pltpu.VMEM((CH,), jnp.float32),
            pltpu.VMEM((RPT,), jnp.float32),
            pltpu.VMEM_SHARED((NPAD,), jnp.float32),
        ],
    )


def _agg_body(nslow, nfast, hs_hbm, src_hbm, dst_hbm, out_hbm, srng, drng,
              bb_v, fb_v, acc_sh, isem, gsem):
    c = lax.axis_index("c")
    s = lax.axis_index("s")

    zero = jnp.zeros((LANES,), jnp.float32)

    def zfill(i, carry):
        for k in range(FDIM // LANES):
            fb_v[i, pl.ds(k * LANES, LANES)] = zero
        return carry

    lax.fori_loop(0, CH, zfill, 0)
    for q in range(RPTA // CH):
        pltpu.sync_copy(fb_v, acc_sh.at[pl.ds(s * RPTA + q * CH, CH)])
    if RPTA % CH:
        pltpu.sync_copy(
            fb_v.at[pl.ds(0, RPTA % CH)],
            acc_sh.at[pl.ds(s * RPTA + (RPTA // CH) * CH, RPTA % CH)])
    plsc.subcore_barrier()

    nch = jnp.where(c == SLOW_CORE, nslow, nfast)

    for r in range(2):
        pltpu.async_copy(src_hbm.at[c, s, r], srng.at[r], isem)
        pltpu.async_copy(dst_hbm.at[c, s, r], drng.at[r], isem)
    pltpu.make_async_copy(src_hbm.at[c, s, 0], srng.at[0], isem).wait()
    pltpu.make_async_copy(dst_hbm.at[c, s, 0], drng.at[0], isem).wait()
    pltpu.async_copy(hs_hbm.at[srng.at[0]], bb_v.at[0], gsem)

    himask = jnp.full((LANES,), -65536, jnp.int32)

    def body(j, carry):
        b = j % NBUF
        pltpu.make_async_copy(
            hs_hbm.at[srng.at[j % NRING]], bb_v.at[b], gsem).wait()
        jc = jnp.minimum(j + 2, nch - 1)
        pltpu.async_copy(src_hbm.at[c, s, jc], srng.at[(j + 2) % NRING], isem)
        pltpu.async_copy(dst_hbm.at[c, s, jc], drng.at[(j + 2) % NRING], isem)
        j1 = jnp.minimum(j + 1, nch - 1)
        pltpu.make_async_copy(
            src_hbm.at[c, s, j1], srng.at[(j + 1) % NRING], isem).wait()
        pltpu.make_async_copy(
            dst_hbm.at[c, s, j1], drng.at[(j + 1) % NRING], isem).wait()
        pltpu.async_copy(
            hs_hbm.at[srng.at[(j + 1) % NRING]], bb_v.at[(j + 1) % NBUF],
            gsem)

        def conv(i, cc):
            for g in range(FDIM // 32):
                w = bb_v[b, i, pl.ds(g * LANES, LANES)]
                lo = jax.lax.bitcast_convert_type(w << 16, jnp.float32)
                hi = jax.lax.bitcast_convert_type(w & himask, jnp.float32)
                fb_v[i, pl.ds(g * 32, LANES)] = lo
                fb_v[i, pl.ds(g * 32 + LANES, LANES)] = hi
            return cc

        lax.fori_loop(0, CH, conv, 0)
        pltpu.sync_copy(fb_v, acc_sh.at[drng.at[j % NRING]], add=True)
        return carry

    lax.fori_loop(0, nch, body, 0)
    pltpu.make_async_copy(
        hs_hbm.at[srng.at[0]], bb_v.at[0], gsem).wait()
    pltpu.make_async_copy(src_hbm.at[c, s, 0], srng.at[0], isem).wait()
    pltpu.make_async_copy(dst_hbm.at[c, s, 0], drng.at[0], isem).wait()
    plsc.subcore_barrier()
    pltpu.sync_copy(acc_sh.at[pl.ds(s * RPTA, RPTA)],
                    out_hbm.at[c, pl.ds(s * RPTA, RPTA)])


@functools.lru_cache(maxsize=None)
def _make_agg(nslow, nfast):
    return pl.kernel(
        functools.partial(_agg_body, nslow, nfast),
        out_type=jax.ShapeDtypeStruct((NC, NPADA, FDIM), jnp.float32),
        mesh=_MESH,
        compiler_params=pltpu.CompilerParams(use_tc_tiling_on_sc=False),
        scratch_types=[
            pltpu.VMEM((NRING, CH), jnp.int32),
            pltpu.VMEM((NRING, CH), jnp.int32),
            pltpu.VMEM((NBUF, CH, FDIM // 2), jnp.int32),
            pltpu.VMEM((CH, FDIM), jnp.float32),
            pltpu.VMEM_SHARED((NPADA, FDIM), jnp.float32),
            pltpu.SemaphoreType.DMA,
            pltpu.SemaphoreType.DMA,
        ],
    )


def _dense1_body(degp_ref, x_ref, w_ref, hs_ref, dinv_ref):
    deg = jnp.sum(degp_ref[...][:, :NODES], axis=0) + 1.0
    dinv = lax.rsqrt(deg)
    dinvb = jnp.broadcast_to(dinv[:, None], (NODES, FDIM))
    dinv_ref[...] = dinvb
    h = jnp.dot(x_ref[...], w_ref[...], preferred_element_type=jnp.float32)
    hs_ref[...] = h * dinvb


def _mid_body(acc_ref, hs_ref, dinv_ref, b_ref, g_ref, be_ref, w2_ref, out_ref):
    dinvb = dinv_ref[...]
    t = (acc_ref[0, :NODES, :] + acc_ref[1, :NODES, :] + hs_ref[...]) * dinvb
    t = t + b_ref[...]
    t = jnp.where(t >= 0.0, t, 0.01 * t)
    mean = jnp.mean(t, axis=0)
    var = jnp.mean((t - mean) ** 2, axis=0)
    t = (t - mean) * lax.rsqrt(var + 1e-5) * g_ref[...] + be_ref[...]
    h2 = jnp.dot(t, w2_ref[...], preferred_element_type=jnp.float32)
    out_ref[...] = h2 * dinvb


def _fin_body(acc_ref, hs_ref, dinv_ref, b_ref, g_ref, be_ref, out_ref):
    t = (acc_ref[0, :NODES, :] + acc_ref[1, :NODES, :] + hs_ref[...]) * dinv_ref[...]
    t = t + b_ref[...]
    mean = jnp.mean(t, axis=0)
    var = jnp.mean((t - mean) ** 2, axis=0)
    out_ref[...] = (t - mean) * lax.rsqrt(var + 1e-5) * g_ref[...] + be_ref[...]


_dense1_call = pl.pallas_call(
    _dense1_body,
    out_shape=[
        jax.ShapeDtypeStruct((NODES, FDIM), jnp.float32),
        jax.ShapeDtypeStruct((NODES, FDIM), jnp.float32),
    ],
)

_mid_call = pl.pallas_call(
    _mid_body,
    out_shape=jax.ShapeDtypeStruct((NODES, FDIM), jnp.float32),
)

_fin_call = pl.pallas_call(
    _fin_body,
    out_shape=jax.ShapeDtypeStruct((NODES, FDIM), jnp.float32),
)


def _pack_bf16(hs):
    hb = hs.astype(jnp.bfloat16).reshape(NODES, FDIM // 32, 2, LANES)
    hb = jnp.transpose(hb, (0, 1, 3, 2))
    return jax.lax.bitcast_convert_type(hb, jnp.int32).reshape(
        NODES, FDIM // 2)


def kernel(x, edge_indices, W1, b1, gamma1, beta1, W2, b2, gamma2, beta2):
    src = edge_indices[0]
    dst = edge_indices[1]
    e = src.shape[0]
    ntot = -(-e // (NS * CH))
    nslow = max(1, (ntot * SLOW_NUM) // SLOW_DEN)
    nfast = ntot - nslow
    nmax = max(nslow, nfast)
    cap_slow = NS * nslow * CH
    cap_fast = NS * nfast * CH
    pad_total = cap_slow + cap_fast - e

    def slab(arr, fill):
        a = jnp.concatenate(
            [arr, jnp.full((pad_total,), fill, arr.dtype)])
        a_sl = a[:cap_slow].reshape(NS, nslow * CH)
        a_fa = a[cap_slow:].reshape(NS, nfast * CH)
        a_sl = jnp.pad(a_sl, ((0, 0), (0, (nmax - nslow) * CH)),
                       constant_values=fill)
        a_fa = jnp.pad(a_fa, ((0, 0), (0, (nmax - nfast) * CH)),
                       constant_values=fill)
        pair = [a_sl, a_fa] if SLOW_CORE == 0 else [a_fa, a_sl]
        return jnp.stack(pair)

    srcp = slab(src, 0).reshape(NC, NS, nmax, CH)
    dstp = slab(dst, NODES).reshape(NC, NS, nmax, CH)

    degp = _make_deg(nslow, nfast)(dstp)
    hs1, dinvb = _dense1_call(degp, x, W1)
    acc1 = _make_agg(nslow, nfast)(_pack_bf16(hs1), srcp, dstp)
    hs2 = _mid_call(acc1, hs1, dinvb, b1, gamma1, beta1, W2)
    acc2 = _make_agg(nslow, nfast)(_pack_bf16(hs2), srcp, dstp)
    out = _fin_call(acc2, hs2, dinvb, b2, gamma2, beta2)
    return out


# speedup vs baseline: 1.5438x; 1.5438x over previous
import functools

import jax
import jax.numpy as jnp
from jax import lax
from jax.experimental import pallas as pl
from jax.experimental.pallas import tpu as pltpu
from jax.experimental.pallas import tpu_sc as plsc

NODES = 10000
FDIM = 128
NC = 2
NS = 16
LANES = 16
CH = 128
RPT = 640
NPAD = NS * RPT
RPTA = 632
NPADA = NS * RPTA
SLOW_CORE = 0
SLOW_NUM, SLOW_DEN = 9, 28
NBUF = 2
NRING = 4

_MESH = plsc.VectorSubcoreMesh(
    core_axis_name="c", subcore_axis_name="s", num_cores=NC, num_subcores=NS)


def _deg_body(nslow, nfast, dst_hbm, out_hbm, dst_v, val_v, zer_v, deg_sh):
    c = lax.axis_index("c")
    s = lax.axis_index("s")
    pltpu.sync_copy(dst_hbm.at[c, s], dst_v)
    one = jnp.ones((LANES,), jnp.float32)
    zero = jnp.zeros((LANES,), jnp.float32)
    for i in range(CH // LANES):
        val_v[pl.ds(i * LANES, LANES)] = one

    def zfill(i, carry):
        zer_v[pl.ds(i * LANES, LANES)] = zero
        return carry

    lax.fori_loop(0, RPT // LANES, zfill, 0)
    pltpu.sync_copy(zer_v, deg_sh.at[pl.ds(s * RPT, RPT)])
    plsc.subcore_barrier()

    nch = jnp.where(c == SLOW_CORE, nslow, nfast)

    def body(j, carry):
        pltpu.sync_copy(val_v, deg_sh.at[dst_v.at[j]], add=True)
        return carry

    lax.fori_loop(0, nch, body, 0)
    plsc.subcore_barrier()
    pltpu.sync_copy(deg_sh.at[pl.ds(s * RPT, RPT)],
                    out_hbm.at[c, pl.ds(s * RPT, RPT)])


@functools.lru_cache(maxsize=None)
def _make_deg(nslow, nfast):
    nmax = max(nslow, nfast)
    return pl.kernel(
        functools.partial(_deg_body, nslow, nfast),
        out_type=jax.ShapeDtypeStruct((NC, NPAD), jnp.float32),
        mesh=_MESH,
        scratch_types=[
            pltpu.VMEM((nmax, CH), jnp.int32),
            pltpu.VMEM((CH,), jnp.float32),
            pltpu.VMEM((RPT,), jnp.float32),
            pltpu.VMEM_SHARED((NPAD,), jnp.float32),
        ],
    )


def _agg_body(nslow, nfast, hs_hbm, src_hbm, dst_hbm, out_hbm, srng, drng,
              buf_v, acc_sh, isem, gsem):
    c = lax.axis_index("c")
    s = lax.axis_index("s")

    zero = jnp.zeros((LANES,), jnp.float32)

    def zfill(i, carry):
        for k in range(FDIM // LANES):
            buf_v[0, i, pl.ds(k * LANES, LANES)] = zero
        return carry

    lax.fori_loop(0, CH, zfill, 0)
    for q in range(RPTA // CH):
        pltpu.sync_copy(buf_v.at[0], acc_sh.at[pl.ds(s * RPTA + q * CH, CH)])
    if RPTA % CH:
        pltpu.sync_copy(
            buf_v.at[0, pl.ds(0, RPTA % CH)],
            acc_sh.at[pl.ds(s * RPTA + (RPTA // CH) * CH, RPTA % CH)])
    plsc.subcore_barrier()

    nch = jnp.where(c == SLOW_CORE, nslow, nfast)

    for r in range(2):
        pltpu.async_copy(src_hbm.at[c, s, r], srng.at[r], isem)
        pltpu.async_copy(dst_hbm.at[c, s, r], drng.at[r], isem)
    pltpu.make_async_copy(src_hbm.at[c, s, 0], srng.at[0], isem).wait()
    pltpu.make_async_copy(dst_hbm.at[c, s, 0], drng.at[0], isem).wait()
    pltpu.async_copy(hs_hbm.at[srng.at[0]], buf_v.at[0], gsem)

    def body(j, carry):
        b = j % NBUF
        pltpu.make_async_copy(
            hs_hbm.at[srng.at[j % NRING]], buf_v.at[b], gsem).wait()
        jc = jnp.minimum(j + 2, nch - 1)
        pltpu.async_copy(src_hbm.at[c, s, jc], srng.at[(j + 2) % NRING], isem)
        pltpu.async_copy(dst_hbm.at[c, s, jc], drng.at[(j + 2) % NRING], isem)
        j1 = jnp.minimum(j + 1, nch - 1)
        pltpu.make_async_copy(
            src_hbm.at[c, s, j1], srng.at[(j + 1) % NRING], isem).wait()
        pltpu.make_async_copy(
            dst_hbm.at[c, s, j1], drng.at[(j + 1) % NRING], isem).wait()
        pltpu.async_copy(
            hs_hbm.at[srng.at[(j + 1) % NRING]], buf_v.at[(j + 1) % NBUF],
            gsem)
        pltpu.sync_copy(buf_v.at[b], acc_sh.at[drng.at[j % NRING]], add=True)
        return carry

    lax.fori_loop(0, nch, body, 0)
    pltpu.make_async_copy(
        hs_hbm.at[srng.at[0]], buf_v.at[0], gsem).wait()
    pltpu.make_async_copy(src_hbm.at[c, s, 0], srng.at[0], isem).wait()
    pltpu.make_async_copy(dst_hbm.at[c, s, 0], drng.at[0], isem).wait()
    plsc.subcore_barrier()
    pltpu.sync_copy(acc_sh.at[pl.ds(s * RPTA, RPTA)],
                    out_hbm.at[c, pl.ds(s * RPTA, RPTA)])


@functools.lru_cache(maxsize=None)
def _make_agg(nslow, nfast):
    return pl.kernel(
        functools.partial(_agg_body, nslow, nfast),
        out_type=jax.ShapeDtypeStruct((NC, NPADA, FDIM), jnp.float32),
        mesh=_MESH,
        scratch_types=[
            pltpu.VMEM((NRING, CH), jnp.int32),
            pltpu.VMEM((NRING, CH), jnp.int32),
            pltpu.VMEM((NBUF, CH, FDIM), jnp.float32),
            pltpu.VMEM_SHARED((NPADA, FDIM), jnp.float32),
            pltpu.SemaphoreType.DMA,
            pltpu.SemaphoreType.DMA,
        ],
    )


def _dense1_body(degp_ref, x_ref, w_ref, hs_ref, dinv_ref):
    deg = jnp.sum(degp_ref[...][:, :NODES], axis=0) + 1.0
    dinv = lax.rsqrt(deg)
    dinvb = jnp.broadcast_to(dinv[:, None], (NODES, FDIM))
    dinv_ref[...] = dinvb
    h = jnp.dot(x_ref[...], w_ref[...], preferred_element_type=jnp.float32)
    hs_ref[...] = h * dinvb


def _mid_body(acc_ref, hs_ref, dinv_ref, b_ref, g_ref, be_ref, w2_ref, out_ref):
    dinvb = dinv_ref[...]
    t = (acc_ref[0, :NODES, :] + acc_ref[1, :NODES, :] + hs_ref[...]) * dinvb
    t = t + b_ref[...]
    t = jnp.where(t >= 0.0, t, 0.01 * t)
    mean = jnp.mean(t, axis=0)
    var = jnp.mean((t - mean) ** 2, axis=0)
    t = (t - mean) * lax.rsqrt(var + 1e-5) * g_ref[...] + be_ref[...]
    h2 = jnp.dot(t, w2_ref[...], preferred_element_type=jnp.float32)
    out_ref[...] = h2 * dinvb


def _fin_body(acc_ref, hs_ref, dinv_ref, b_ref, g_ref, be_ref, out_ref):
    t = (acc_ref[0, :NODES, :] + acc_ref[1, :NODES, :] + hs_ref[...]) * dinv_ref[...]
    t = t + b_ref[...]
    mean = jnp.mean(t, axis=0)
    var = jnp.mean((t - mean) ** 2, axis=0)
    out_ref[...] = (t - mean) * lax.rsqrt(var + 1e-5) * g_ref[...] + be_ref[...]


_dense1_call = pl.pallas_call(
    _dense1_body,
    out_shape=[
        jax.ShapeDtypeStruct((NODES, FDIM), jnp.float32),
        jax.ShapeDtypeStruct((NODES, FDIM), jnp.float32),
    ],
)

_mid_call = pl.pallas_call(
    _mid_body,
    out_shape=jax.ShapeDtypeStruct((NODES, FDIM), jnp.float32),
)

_fin_call = pl.pallas_call(
    _fin_body,
    out_shape=jax.ShapeDtypeStruct((NODES, FDIM), jnp.float32),
)


def kernel(x, edge_indices, W1, b1, gamma1, beta1, W2, b2, gamma2, beta2):
    src = edge_indices[0]
    dst = edge_indices[1]
    e = src.shape[0]
    ntot = -(-e // (NS * CH))
    nslow = max(1, (ntot * SLOW_NUM) // SLOW_DEN)
    nfast = ntot - nslow
    nmax = max(nslow, nfast)
    cap_slow = NS * nslow * CH
    cap_fast = NS * nfast * CH
    pad_total = cap_slow + cap_fast - e

    def slab(arr, fill):
        a = jnp.concatenate(
            [arr, jnp.full((pad_total,), fill, arr.dtype)])
        a_sl = a[:cap_slow].reshape(NS, nslow * CH)
        a_fa = a[cap_slow:].reshape(NS, nfast * CH)
        a_sl = jnp.pad(a_sl, ((0, 0), (0, (nmax - nslow) * CH)),
                       constant_values=fill)
        a_fa = jnp.pad(a_fa, ((0, 0), (0, (nmax - nfast) * CH)),
                       constant_values=fill)
        pair = [a_sl, a_fa] if SLOW_CORE == 0 else [a_fa, a_sl]
        return jnp.stack(pair)

    srcp = slab(src, 0).reshape(NC, NS, nmax, CH)
    dstp = slab(dst, NODES).reshape(NC, NS, nmax, CH)

    degp = _make_deg(nslow, nfast)(dstp)
    hs1, dinvb = _dense1_call(degp, x, W1)
    acc1 = _make_agg(nslow, nfast)(hs1, srcp, dstp)
    hs2 = _mid_call(acc1, hs1, dinvb, b1, gamma1, beta1, W2)
    acc2 = _make_agg(nslow, nfast)(hs2, srcp, dstp)
    out = _fin_call(acc2, hs2, dinvb, b2, gamma2, beta2)
    return out
